# Initial kernel scaffold; baseline (speedup 1.0000x reference)
import functools

import jax
import jax.numpy as jnp
from jax import lax
from jax.experimental import pallas as pl
from jax.experimental.pallas import tpu as pltpu
from jax.experimental.pallas import tpu_sc as plsc

NC = 2
NS = 16
NW = NC * NS
L = 16
CH = 128
D = 128


def _make_deg_kernel(npad, cpw):
    rows = npad // 128
    rpt = rows // NS
    mesh = plsc.VectorSubcoreMesh(core_axis_name="c", subcore_axis_name="s")

    @functools.partial(
        pl.kernel,
        out_type=jax.ShapeDtypeStruct((NC, rows, 128), jnp.float32),
        mesh=mesh,
        scratch_types=[
            pltpu.VMEM((cpw, CH), jnp.int32),
            pltpu.VMEM((rows, 128), jnp.float32),
            pltpu.VMEM((rows,), jnp.int32),
            pltpu.VMEM_SHARED((rows, 128), jnp.float32),
        ],
    )
    def deg_kernel(dst_hbm, out_hbm, dstv, dloc, identv, shared):
        c = lax.axis_index("c")
        s = lax.axis_index("s")
        wid = s * NC + c
        pltpu.sync_copy(dst_hbm.at[wid], dstv)

        zeros16 = jnp.zeros((L,), jnp.float32)

        def zero_row(j, carry):
            for kk in range(128 // L):
                dloc[j, pl.ds(kk * L, L)] = zeros16
            return carry

        lax.fori_loop(0, rows, zero_row, None)
        my_rows = pl.ds(s * rpt, rpt)
        pltpu.sync_copy(dloc.at[my_rows], shared.at[my_rows])
        iota = lax.iota(jnp.int32, L)
        for i in range(rows // L):
            identv[pl.ds(i * L, L)] = iota + i * L

        ones16 = jnp.ones((L,), jnp.float32)

        def count(j, carry):
            for kk in range(CH // L):
                idx = dstv[j, pl.ds(kk * L, L)]
                r = lax.shift_right_logical(idx, 7)
                col = lax.bitwise_and(idx, 127)
                plsc.addupdate_scatter(dloc, [r, col], ones16)
            return carry

        lax.fori_loop(0, cpw, count, None)
        plsc.subcore_barrier()
        pltpu.sync_copy(dloc, shared.at[identv], add=True)
        plsc.subcore_barrier()
        pltpu.sync_copy(shared.at[my_rows], out_hbm.at[c, my_rows])

    return deg_kernel


def _make_agg_kernel(npad, cpw):
    rpt = npad // NS
    mesh = plsc.VectorSubcoreMesh(core_axis_name="c", subcore_axis_name="s")

    @functools.partial(
        pl.kernel,
        out_type=jax.ShapeDtypeStruct((NC, npad, D), jnp.float32),
        mesh=mesh,
        scratch_types=[
            pltpu.VMEM((cpw, CH), jnp.int32),
            pltpu.VMEM((cpw, CH), jnp.int32),
            pltpu.VMEM((2, CH, D), jnp.float32),
            pltpu.VMEM_SHARED((npad, D), jnp.float32),
            pltpu.SemaphoreType.DMA,
            pltpu.SemaphoreType.DMA,
        ],
    )
    def agg_kernel(h_hbm, src_hbm, dst_hbm, zz_hbm, out_hbm,
                   srcv, dstv, rbuf, acc, sem0, sem1):
        c = lax.axis_index("c")
        s = lax.axis_index("s")
        wid = s * NC + c
        pltpu.sync_copy(src_hbm.at[wid], srcv)
        pltpu.sync_copy(dst_hbm.at[wid], dstv)
        pltpu.async_copy(h_hbm.at[srcv.at[0]], rbuf.at[0], sem0)
        pltpu.async_copy(h_hbm.at[srcv.at[1]], rbuf.at[1], sem1)
        my_rows = pl.ds(s * rpt, rpt)

        @pl.when(c == 0)
        def _():
            pltpu.sync_copy(h_hbm.at[my_rows], acc.at[my_rows])

        @pl.when(c != 0)
        def _():
            pltpu.sync_copy(zz_hbm, acc.at[my_rows])

        plsc.subcore_barrier()

        sems = (sem0, sem1)

        def step(i, carry):
            j0 = i * 2
            for b in range(2):
                j = j0 + b
                pltpu.make_async_copy(
                    h_hbm.at[srcv.at[j]], rbuf.at[b], sems[b]).wait()
                pltpu.sync_copy(rbuf.at[b], acc.at[dstv.at[j]], add=True)

                @pl.when(j + 2 < cpw)
                def _():
                    pltpu.async_copy(
                        h_hbm.at[srcv.at[j + 2]], rbuf.at[b], sems[b])
            return carry

        lax.fori_loop(0, cpw // 2, step, None)
        plsc.subcore_barrier()
        pltpu.sync_copy(acc.at[my_rows], out_hbm.at[c, my_rows])

    return agg_kernel


def _mm_pre(x, dinv, W1, npad):
    BR = 1280

    def body(x_ref, d_ref, w_ref, o_ref):
        o_ref[...] = jnp.dot(x_ref[...] * d_ref[...], w_ref[...],
                             preferred_element_type=jnp.float32)

    return pl.pallas_call(
        body,
        grid=(npad // BR,),
        in_specs=[
            pl.BlockSpec((BR, D), lambda i: (i, 0)),
            pl.BlockSpec((BR, 1), lambda i: (i, 0)),
            pl.BlockSpec((D, D), lambda i: (0, 0)),
        ],
        out_specs=pl.BlockSpec((BR, D), lambda i: (i, 0)),
        out_shape=jax.ShapeDtypeStruct((npad, D), jnp.float32),
    )(x, dinv, W1)


def _mm_mid(p, dinv, b, W2, npad):
    BR = 1280

    def body(p_ref, d_ref, b_ref, w_ref, o_ref):
        t = (p_ref[0] + p_ref[1]) * d_ref[...] + b_ref[...]
        h1 = jnp.maximum(t, 0.0)
        o_ref[...] = jnp.dot(h1 * d_ref[...], w_ref[...],
                             preferred_element_type=jnp.float32)

    return pl.pallas_call(
        body,
        grid=(npad // BR,),
        in_specs=[
            pl.BlockSpec((NC, BR, D), lambda i: (0, i, 0)),
            pl.BlockSpec((BR, 1), lambda i: (i, 0)),
            pl.BlockSpec((1, D), lambda i: (0, 0)),
            pl.BlockSpec((D, D), lambda i: (0, 0)),
        ],
        out_specs=pl.BlockSpec((BR, D), lambda i: (i, 0)),
        out_shape=jax.ShapeDtypeStruct((npad, D), jnp.float32),
    )(p, dinv, b, W2)


def _mm_post(q, dinv, b, Wo, bo, npad, ncls):
    BR = 1280

    def body(q_ref, d_ref, b_ref, w_ref, bo_ref, o_ref):
        t = (q_ref[0] + q_ref[1]) * d_ref[...] + b_ref[...]
        h2 = jnp.maximum(t, 0.0)
        lg = jnp.dot(h2, w_ref[...],
                     preferred_element_type=jnp.float32) + bo_ref[...]
        colmask = lax.broadcasted_iota(jnp.int32, (BR, D), 1) < ncls
        z = jnp.where(colmask, lg, -jnp.inf)
        m = jnp.max(z, axis=1, keepdims=True)
        e = jnp.where(colmask, jnp.exp(z - m), 0.0)
        o_ref[...] = e / jnp.sum(e, axis=1, keepdims=True)

    return pl.pallas_call(
        body,
        grid=(npad // BR,),
        in_specs=[
            pl.BlockSpec((NC, BR, D), lambda i: (0, i, 0)),
            pl.BlockSpec((BR, 1), lambda i: (i, 0)),
            pl.BlockSpec((1, D), lambda i: (0, 0)),
            pl.BlockSpec((D, D), lambda i: (0, 0)),
            pl.BlockSpec((1, D), lambda i: (0, 0)),
        ],
        out_specs=pl.BlockSpec((BR, D), lambda i: (i, 0)),
        out_shape=jax.ShapeDtypeStruct((npad, D), jnp.float32),
    )(q, dinv, b, Wo, bo)


def kernel(x, edge_index, W1, b1, W2, b2, Wout, bout):
    n, d = x.shape
    assert d == D
    e = edge_index.shape[1]
    ncls = Wout.shape[1]
    npad = -(-n // 2048) * 2048
    cpw = -(-e // (NW * CH))
    cpw += cpw % 2
    epad = NW * cpw * CH
    pad = epad - e

    src = edge_index[0].astype(jnp.int32)
    dst = edge_index[1].astype(jnp.int32)
    junk = n + (jnp.arange(pad, dtype=jnp.int32) % (npad - n))
    src_p = jnp.concatenate([src, jnp.zeros((pad,), jnp.int32)]
                            ).reshape(NW, cpw, CH)
    dst_p = jnp.concatenate([dst, junk]).reshape(NW, cpw, CH)
    x_p = jnp.concatenate([x, jnp.zeros((npad - n, D), x.dtype)])
    zz = jnp.zeros((npad // NS, D), jnp.float32)

    deg_parts = _make_deg_kernel(npad, cpw)(dst_p)
    cnt = deg_parts[0] + deg_parts[1]
    dinv = lax.rsqrt(cnt.reshape(npad) + 1.0).reshape(npad, 1)

    agg = _make_agg_kernel(npad, cpw)
    h1p = _mm_pre(x_p, dinv, W1, npad)
    p = agg(h1p, src_p, dst_p, zz)
    h2p = _mm_mid(p, dinv, b1.reshape(1, D), W2, npad)
    q = agg(h2p, src_p, dst_p, zz)
    Wo = jnp.concatenate([Wout, jnp.zeros((D, D - ncls), Wout.dtype)], axis=1)
    bo = jnp.concatenate([bout, jnp.zeros((D - ncls,), bout.dtype)]
                         ).reshape(1, D)
    probs = _mm_post(q, dinv, b2.reshape(1, D), Wo, bo, npad, ncls)
    return probs[:n, :ncls]


# speedup vs baseline: 8.9577x; 8.9577x over previous
import functools

import jax
import jax.numpy as jnp
from jax import lax
from jax.experimental import pallas as pl
from jax.experimental.pallas import tpu as pltpu
from jax.experimental.pallas import tpu_sc as plsc

NC = 2
NS = 16
NW = NC * NS
L = 16
CH = 128
D = 128


def _make_deg_kernel(npad, cpw):
    rpt = npad // NS
    mesh = plsc.VectorSubcoreMesh(core_axis_name="c", subcore_axis_name="s")

    @functools.partial(
        pl.kernel,
        out_type=jax.ShapeDtypeStruct((NC, npad, L), jnp.float32),
        mesh=mesh,
        scratch_types=[
            pltpu.VMEM((cpw, CH), jnp.int32),
            pltpu.VMEM((CH, L), jnp.float32),
            pltpu.VMEM_SHARED((npad, L), jnp.float32),
        ],
    )
    def deg_kernel(dst_hbm, zz_hbm, out_hbm, dstv, onesv, shared):
        c = lax.axis_index("c")
        s = lax.axis_index("s")
        wid = s * NC + c
        pltpu.sync_copy(dst_hbm.at[wid], dstv)
        ones16 = jnp.ones((L,), jnp.float32)

        def fill(j, carry):
            onesv[j, pl.ds(0, L)] = ones16
            return carry

        lax.fori_loop(0, CH, fill, None)
        my_rows = pl.ds(s * rpt, rpt)
        pltpu.sync_copy(zz_hbm, shared.at[my_rows])
        plsc.subcore_barrier()

        def count(j, carry):
            pltpu.sync_copy(onesv, shared.at[dstv.at[j]], add=True)
            return carry

        lax.fori_loop(0, cpw, count, None)
        plsc.subcore_barrier()
        pltpu.sync_copy(shared.at[my_rows], out_hbm.at[c, my_rows])

    return deg_kernel


def _make_agg_kernel(npad, cpw, hw):
    rpt = npad // NS
    mesh = plsc.VectorSubcoreMesh(core_axis_name="c", subcore_axis_name="s")

    @functools.partial(
        pl.kernel,
        out_type=jax.ShapeDtypeStruct((NC, npad, hw), jnp.float32),
        mesh=mesh,
        scratch_types=[
            pltpu.VMEM((cpw, CH), jnp.int32),
            pltpu.VMEM((cpw, CH), jnp.int32),
            pltpu.VMEM((2, CH, hw), jnp.float32),
            pltpu.VMEM_SHARED((npad, hw), jnp.float32),
            pltpu.SemaphoreType.DMA,
            pltpu.SemaphoreType.DMA,
        ],
        compiler_params=pltpu.CompilerParams(use_tc_tiling_on_sc=False),
    )
    def agg_kernel(h_hbm, src_hbm, dst_hbm, zz_hbm, out_hbm,
                   srcv, dstv, rbuf, acc, sem0, sem1):
        c = lax.axis_index("c")
        s = lax.axis_index("s")
        wid = s * NC + c
        pltpu.sync_copy(src_hbm.at[wid], srcv)
        pltpu.sync_copy(dst_hbm.at[wid], dstv)
        pltpu.async_copy(h_hbm.at[srcv.at[0]], rbuf.at[0], sem0)
        pltpu.async_copy(h_hbm.at[srcv.at[1]], rbuf.at[1], sem1)
        my_rows = pl.ds(s * rpt, rpt)

        @pl.when(c == 0)
        def _():
            pltpu.sync_copy(h_hbm.at[my_rows], acc.at[my_rows])

        @pl.when(c != 0)
        def _():
            pltpu.sync_copy(zz_hbm, acc.at[my_rows])

        plsc.subcore_barrier()

        sems = (sem0, sem1)

        def step(i, carry):
            j0 = i * 2
            for b in range(2):
                j = j0 + b
                pltpu.make_async_copy(
                    h_hbm.at[srcv.at[j]], rbuf.at[b], sems[b]).wait()
                pltpu.sync_copy(rbuf.at[b], acc.at[dstv.at[j]], add=True)

                @pl.when(j + 2 < cpw)
                def _():
                    pltpu.async_copy(
                        h_hbm.at[srcv.at[j + 2]], rbuf.at[b], sems[b])
            return carry

        lax.fori_loop(0, cpw // 2, step, None)
        plsc.subcore_barrier()
        pltpu.sync_copy(acc.at[my_rows], out_hbm.at[c, my_rows])

    return agg_kernel


def _mm_pre(x, dinv, W1, npad, hw):
    BR = 1280

    def body(x_ref, d_ref, w_ref, o_lo, o_hi):
        h = jnp.dot(x_ref[...] * d_ref[...], w_ref[...],
                    preferred_element_type=jnp.float32)
        o_lo[...] = h[:, :hw]
        o_hi[...] = h[:, hw:]

    return pl.pallas_call(
        body,
        grid=(npad // BR,),
        in_specs=[
            pl.BlockSpec((BR, D), lambda i: (i, 0)),
            pl.BlockSpec((BR, 1), lambda i: (i, 0)),
            pl.BlockSpec((D, D), lambda i: (0, 0)),
        ],
        out_specs=[pl.BlockSpec((BR, hw), lambda i: (i, 0)),
                   pl.BlockSpec((BR, hw), lambda i: (i, 0))],
        out_shape=[jax.ShapeDtypeStruct((npad, hw), jnp.float32),
                   jax.ShapeDtypeStruct((npad, hw), jnp.float32)],
    )(x, dinv, W1)


def _mm_mid(p_lo, p_hi, dinv, b, W2, npad, hw):
    BR = 1280

    def body(pl_ref, ph_ref, d_ref, b_ref, w_ref, o_lo, o_hi):
        agg = jnp.concatenate([pl_ref[0] + pl_ref[1],
                               ph_ref[0] + ph_ref[1]], axis=1)
        t = agg * d_ref[...] + b_ref[...]
        h1 = jnp.maximum(t, 0.0)
        h = jnp.dot(h1 * d_ref[...], w_ref[...],
                    preferred_element_type=jnp.float32)
        o_lo[...] = h[:, :hw]
        o_hi[...] = h[:, hw:]

    return pl.pallas_call(
        body,
        grid=(npad // BR,),
        in_specs=[
            pl.BlockSpec((NC, BR, hw), lambda i: (0, i, 0)),
            pl.BlockSpec((NC, BR, hw), lambda i: (0, i, 0)),
            pl.BlockSpec((BR, 1), lambda i: (i, 0)),
            pl.BlockSpec((1, D), lambda i: (0, 0)),
            pl.BlockSpec((D, D), lambda i: (0, 0)),
        ],
        out_specs=[pl.BlockSpec((BR, hw), lambda i: (i, 0)),
                   pl.BlockSpec((BR, hw), lambda i: (i, 0))],
        out_shape=[jax.ShapeDtypeStruct((npad, hw), jnp.float32),
                   jax.ShapeDtypeStruct((npad, hw), jnp.float32)],
    )(p_lo, p_hi, dinv, b, W2)


def _mm_post(q_lo, q_hi, dinv, b, Wo, bo, npad, ncls, hw):
    BR = 1280

    def body(ql_ref, qh_ref, d_ref, b_ref, w_ref, bo_ref, o_ref):
        agg = jnp.concatenate([ql_ref[0] + ql_ref[1],
                               qh_ref[0] + qh_ref[1]], axis=1)
        t = agg * d_ref[...] + b_ref[...]
        h2 = jnp.maximum(t, 0.0)
        lg = jnp.dot(h2, w_ref[...],
                     preferred_element_type=jnp.float32) + bo_ref[...]
        colmask = lax.broadcasted_iota(jnp.int32, (BR, D), 1) < ncls
        z = jnp.where(colmask, lg, -jnp.inf)
        m = jnp.max(z, axis=1, keepdims=True)
        e = jnp.where(colmask, jnp.exp(z - m), 0.0)
        o_ref[...] = e / jnp.sum(e, axis=1, keepdims=True)

    return pl.pallas_call(
        body,
        grid=(npad // BR,),
        in_specs=[
            pl.BlockSpec((NC, BR, hw), lambda i: (0, i, 0)),
            pl.BlockSpec((NC, BR, hw), lambda i: (0, i, 0)),
            pl.BlockSpec((BR, 1), lambda i: (i, 0)),
            pl.BlockSpec((1, D), lambda i: (0, 0)),
            pl.BlockSpec((D, D), lambda i: (0, 0)),
            pl.BlockSpec((1, D), lambda i: (0, 0)),
        ],
        out_specs=pl.BlockSpec((BR, D), lambda i: (i, 0)),
        out_shape=jax.ShapeDtypeStruct((npad, D), jnp.float32),
    )(q_lo, q_hi, dinv, b, Wo, bo)


def kernel(x, edge_index, W1, b1, W2, b2, Wout, bout):
    n, d = x.shape
    assert d == D
    e = edge_index.shape[1]
    ncls = Wout.shape[1]
    npad = -(-n // 2048) * 2048
    cpw = -(-e // (NW * CH))
    cpw += cpw % 2
    epad = NW * cpw * CH
    pad = epad - e

    src = edge_index[0].astype(jnp.int32)
    dst = edge_index[1].astype(jnp.int32)
    junk = n + (jnp.arange(pad, dtype=jnp.int32) % (npad - n))
    src_p = jnp.concatenate([src, jnp.zeros((pad,), jnp.int32)]
                            ).reshape(NW, cpw, CH)
    dst_p = jnp.concatenate([dst, junk]).reshape(NW, cpw, CH)
    hw = D // 2
    x_p = jnp.concatenate([x, jnp.zeros((npad - n, D), x.dtype)])
    zz = jnp.zeros((npad // NS, hw), jnp.float32)

    zz16 = jnp.zeros((npad // NS, L), jnp.float32)
    deg_parts = _make_deg_kernel(npad, cpw)(dst_p, zz16)
    cnt = deg_parts[0, :, 0] + deg_parts[1, :, 0]
    dinv = lax.rsqrt(cnt + 1.0).reshape(npad, 1)

    agg = _make_agg_kernel(npad, cpw, hw)
    h1_lo, h1_hi = _mm_pre(x_p, dinv, W1, npad, hw)
    p_lo = agg(h1_lo, src_p, dst_p, zz)
    p_hi = agg(h1_hi, src_p, dst_p, zz)
    h2_lo, h2_hi = _mm_mid(p_lo, p_hi, dinv, b1.reshape(1, D), W2, npad, hw)
    q_lo = agg(h2_lo, src_p, dst_p, zz)
    q_hi = agg(h2_hi, src_p, dst_p, zz)
    Wo = jnp.concatenate([Wout, jnp.zeros((D, D - ncls), Wout.dtype)], axis=1)
    bo = jnp.concatenate([bout, jnp.zeros((D - ncls,), bout.dtype)]
                         ).reshape(1, D)
    probs = _mm_post(q_lo, q_hi, dinv, b2.reshape(1, D), Wo, bo, npad, ncls, hw)
    return probs[:n, :ncls]
